# baseline (speedup 1.0000x reference)
import jax
import jax.numpy as jnp
from jax.experimental import pallas as pl
from jax.experimental.pallas import tpu as pltpu

_H, _W, _KS = 4096, 4096, 4
_B = 16
_HB = _H // _B
_N = 8192
_NSLAB = 32


def _build_kernel(order_ref, cols_ref, starts_ref, x_ref, sb_ref, out_ref,
                  vres_ref, t_ref):
    step = pl.program_id(0)

    @pl.when(step < _NSLAB)
    def _phase_a():
        m = step
        for g in range(8):
            chunk = x_ref[2 * g:2 * g + 2].reshape(256, 128)
            tc = jnp.swapaxes(chunk, 0, 1)
            t_ref[0, :, pl.ds(g * 128, 128)] = tc[:, 0:128]
            t_ref[1, :, pl.ds(g * 128, 128)] = tc[:, 128:256]
        for jj in range(2):
            for p in range(2):
                for ks in range(4):
                    tt = jj * 8 + p * 4 + ks
                    y = jax.lax.dot_general(
                        t_ref[p], sb_ref[tt],
                        (((1,), (0,)), ((), ())),
                        preferred_element_type=jnp.float32,
                    )
                    vres_ref[pl.ds(m * 256 + jj * 128, 128), p * 4 + ks, :] = y

    @pl.when(step >= _NSLAB)
    def _phase_b():
        i = step - _NSLAB
        out_ref[...] = jnp.zeros_like(out_ref)

        lane = jax.lax.broadcasted_iota(jnp.int32, (8, 128), 1)

        def scatter_one(t):
            o = order_ref[t]
            c = cols_ref[t]
            tq = (c // 8) * 4
            cm8 = c % 8
            s16 = cm8 * 16
            v = vres_ref[o]
            keep = (lane >= s16) & (lane < s16 + 16)
            for g in range(8):
                sh = ((cm8 - g) % 8) * 16
                rolled = pltpu.roll(v, sh, 1)
                upd = jnp.where(keep, rolled, 0.0)
                out_ref[2 * g, pl.ds(tq, 4), :] += upd[0:4]
                out_ref[2 * g + 1, pl.ds(tq, 4), :] += upd[4:8]

        start = starts_ref[i]
        end = starts_ref[i + 1]
        n = end - start

        def body16(t16, carry):
            t = start + 16 * t16
            for u in range(16):
                scatter_one(t + u)
            return carry

        jax.lax.fori_loop(0, n // 16, body16, 0)

        def body1(t, carry):
            scatter_one(t)
            return carry

        jax.lax.fori_loop(start + (n // 16) * 16, end, body1, 0)


def _selection_matrices():
    src = jnp.arange(1024, dtype=jnp.int32)[:, None]
    dst = jnp.arange(128, dtype=jnp.int32)[None, :]
    g_s, rem = src // 128, src % 128
    k_s, jj_s, ks_s = rem // 8, (rem % 8) // 4, rem % 4
    mats = []
    for jj in range(2):
        for p in range(2):
            for ks in range(4):
                cond = (
                    (g_s == dst // 16) & (k_s == dst % 16)
                    & (jj_s == jj) & (ks_s == ks)
                )
                mats.append(cond.astype(jnp.float32))
    return jnp.stack(mats)


def kernel(block_indices, block_values):
    rows = block_indices[:, 0].astype(jnp.int32)
    cols = block_indices[:, 1].astype(jnp.int32)
    iota = jnp.arange(_N, dtype=jnp.int32)
    keys = jnp.sort((rows << 21) | (cols << 13) | iota)
    order = keys & 0x1FFF
    sorted_cols = (keys >> 13) & 0xFF
    starts = jnp.sum(
        rows[None, :] < jnp.arange(_HB + 1, dtype=jnp.int32)[:, None],
        axis=1,
        dtype=jnp.int32,
    )

    x4 = (
        block_values.reshape(64, 128, _B, _B, _KS)
        .transpose(2, 3, 0, 4, 1)
        .reshape(_B, _B, 256, 128)
    )
    sb = _selection_matrices()

    out = pl.pallas_call(
        _build_kernel,
        grid_spec=pltpu.PrefetchScalarGridSpec(
            num_scalar_prefetch=3,
            grid=(_NSLAB + _HB,),
            in_specs=[
                pl.BlockSpec(
                    (_B, _B, 2 * _KS, 128),
                    lambda s, *_: (0, 0, jnp.minimum(s, _NSLAB - 1), 0),
                ),
                pl.BlockSpec((16, 1024, 128), lambda s, *_: (0, 0, 0)),
            ],
            out_specs=pl.BlockSpec(
                (_B, 128, 128),
                lambda s, *_: (jnp.maximum(s - _NSLAB, 0), 0, 0),
            ),
            scratch_shapes=[
                pltpu.VMEM((_N, 8, 128), jnp.float32),
                pltpu.VMEM((2, 128, 1024), jnp.float32),
            ],
        ),
        out_shape=jax.ShapeDtypeStruct((_H, 128, 128), jnp.float32),
        compiler_params=pltpu.CompilerParams(
            dimension_semantics=("arbitrary",),
            vmem_limit_bytes=56 * 1024 * 1024,
        ),
    )(order, sorted_cols, starts, x4, sb)

    out = out.reshape(_H, 32, _KS, 128)
    out = out.transpose(0, 1, 3, 2)
    return out.reshape(_H, _W, _KS)


# speedup vs baseline: 1.0456x; 1.0456x over previous
import jax
import jax.numpy as jnp
from jax.experimental import pallas as pl
from jax.experimental.pallas import tpu as pltpu

_H, _W, _KS = 4096, 4096, 4
_B = 16
_HB = _H // _B
_N = 8192
_NSLAB = 32


def _build_kernel(order_ref, cols_ref, starts_ref, x_ref, sb_ref, out_ref,
                  vres_ref, t_ref):
    step = pl.program_id(0)

    @pl.when(step < _NSLAB)
    def _phase_a():
        m = step
        for g in range(8):
            chunk = x_ref[2 * g:2 * g + 2].reshape(256, 128)
            tc = jnp.swapaxes(chunk, 0, 1)
            t_ref[0, :, pl.ds(g * 128, 128)] = tc[:, 0:128]
            t_ref[1, :, pl.ds(g * 128, 128)] = tc[:, 128:256]
        for jj in range(2):
            for p in range(2):
                for ks in range(4):
                    tt = jj * 8 + p * 4 + ks
                    y = jax.lax.dot_general(
                        t_ref[p], sb_ref[tt],
                        (((1,), (0,)), ((), ())),
                        preferred_element_type=jnp.float32,
                    )
                    vres_ref[pl.ds(m * 256 + jj * 128, 128), p * 4 + ks, :] = y

    @pl.when(step >= _NSLAB)
    def _phase_b():
        i = step - _NSLAB
        out_ref[...] = jnp.zeros_like(out_ref)

        lane = jax.lax.broadcasted_iota(jnp.int32, (8, 128), 1)

        def scatter_one(t):
            o = order_ref[t]
            c = cols_ref[t]
            tq = (c // 8) * 4
            cm8 = c % 8
            s16 = cm8 * 16
            v = vres_ref[o]
            keep = (lane >= s16) & (lane < s16 + 16)
            for g in range(8):
                sh = ((cm8 - g) % 8) * 16
                rolled = pltpu.roll(v, sh, 1)
                upd = jnp.where(keep, rolled, 0.0)
                out_ref[2 * g, pl.ds(tq, 4), :] += upd[0:4]
                out_ref[2 * g + 1, pl.ds(tq, 4), :] += upd[4:8]

        start = starts_ref[i]
        end = starts_ref[i + 1]
        n = end - start

        def body8(t8, carry):
            t = start + 8 * t8
            for u in range(8):
                scatter_one(t + u)
            return carry

        jax.lax.fori_loop(0, n // 8, body8, 0)

        def body1(t, carry):
            scatter_one(t)
            return carry

        jax.lax.fori_loop(start + (n // 8) * 8, end, body1, 0)


def _selection_matrices():
    src = jnp.arange(1024, dtype=jnp.int32)[:, None]
    dst = jnp.arange(128, dtype=jnp.int32)[None, :]
    g_s, rem = src // 128, src % 128
    k_s, jj_s, ks_s = rem // 8, (rem % 8) // 4, rem % 4
    mats = []
    for jj in range(2):
        for p in range(2):
            for ks in range(4):
                cond = (
                    (g_s == dst // 16) & (k_s == dst % 16)
                    & (jj_s == jj) & (ks_s == ks)
                )
                mats.append(cond.astype(jnp.float32))
    return jnp.stack(mats)


def kernel(block_indices, block_values):
    rows = block_indices[:, 0].astype(jnp.int32)
    cols = block_indices[:, 1].astype(jnp.int32)
    iota = jnp.arange(_N, dtype=jnp.int32)
    keys = jnp.sort((rows << 21) | (cols << 13) | iota)
    order = keys & 0x1FFF
    sorted_cols = (keys >> 13) & 0xFF
    starts = jnp.sum(
        rows[None, :] < jnp.arange(_HB + 1, dtype=jnp.int32)[:, None],
        axis=1,
        dtype=jnp.int32,
    )

    x4 = (
        block_values.reshape(64, 128, _B, _B, _KS)
        .transpose(2, 3, 0, 4, 1)
        .reshape(_B, _B, 256, 128)
    )
    sb = _selection_matrices()

    out = pl.pallas_call(
        _build_kernel,
        grid_spec=pltpu.PrefetchScalarGridSpec(
            num_scalar_prefetch=3,
            grid=(_NSLAB + _HB,),
            in_specs=[
                pl.BlockSpec(
                    (_B, _B, 2 * _KS, 128),
                    lambda s, *_: (0, 0, jnp.minimum(s, _NSLAB - 1), 0),
                ),
                pl.BlockSpec((16, 1024, 128), lambda s, *_: (0, 0, 0)),
            ],
            out_specs=pl.BlockSpec(
                (_B, 128, 128),
                lambda s, *_: (jnp.maximum(s - _NSLAB, 0), 0, 0),
            ),
            scratch_shapes=[
                pltpu.VMEM((_N, 8, 128), jnp.float32),
                pltpu.VMEM((2, 128, 1024), jnp.float32),
            ],
        ),
        out_shape=jax.ShapeDtypeStruct((_H, 128, 128), jnp.float32),
        compiler_params=pltpu.CompilerParams(
            dimension_semantics=("arbitrary",),
            vmem_limit_bytes=56 * 1024 * 1024,
        ),
    )(order, sorted_cols, starts, x4, sb)

    out = out.reshape(_H, 32, _KS, 128)
    out = out.transpose(0, 1, 3, 2)
    return out.reshape(_H, _W, _KS)
